# Initial kernel scaffold; baseline (speedup 1.0000x reference)
import jax
import jax.numpy as jnp
from jax.experimental import pallas as pl


def kernel(x_a, x_b, W1, b1, W2, b2, Wh, bh, Wp, bp, edge_index_a, edge_index_b, states):
    raise NotImplementedError("write your pallas kernel here")


# speedup vs baseline: 12.2249x; 12.2249x over previous
import functools

import jax
import jax.numpy as jnp
from jax import lax
from jax.experimental import pallas as pl
from jax.experimental.pallas import tpu as pltpu
from jax.experimental.pallas import tpu_sc as plsc

_NC = 2
_NS = 16
_CH = 80


def _sc_mesh():
    return plsc.VectorSubcoreMesh(
        core_axis_name="c", subcore_axis_name="s", num_cores=_NC, num_subcores=_NS
    )


def _make_deg_kernel(NP, E):
    rows_pt = NP // _NS
    epc = E // _NS
    n_chunks = epc // _CH

    @functools.partial(
        pl.kernel,
        mesh=_sc_mesh(),
        out_type=jax.ShapeDtypeStruct((2 * NP, 16), jnp.float32),
        scratch_types=[
            pltpu.VMEM((_CH,), jnp.int32),
            pltpu.VMEM((_CH, 16), jnp.float32),
            pltpu.VMEM_SHARED((NP, 16), jnp.float32),
        ],
    )
    def deg_kernel(dst_hbm, ones_hbm, init_hbm, out_hbm, idx_v, ones_v, acc_sh):
        cid = lax.axis_index("c")
        sid = lax.axis_index("s")
        r0 = sid * rows_pt
        pltpu.sync_copy(
            init_hbm.at[pl.ds(r0, rows_pt)], acc_sh.at[pl.ds(r0, rows_pt)]
        )
        pltpu.sync_copy(ones_hbm, ones_v)
        plsc.subcore_barrier()

        def body(i, carry):
            base = cid * E + sid * epc + i * _CH
            pltpu.sync_copy(dst_hbm.at[pl.ds(base, _CH)], idx_v)
            pltpu.sync_copy(ones_v, acc_sh.at[idx_v], add=True)
            return carry

        lax.fori_loop(0, n_chunks, body, 0)
        plsc.subcore_barrier()
        pltpu.sync_copy(
            acc_sh.at[pl.ds(r0, rows_pt)],
            out_hbm.at[pl.ds(cid * NP + r0, rows_pt)],
        )

    return deg_kernel


def _make_agg_kernel(NP, E, D):
    rows_pt = NP // _NS
    epc = E // _NS
    n_chunks = epc // _CH

    @functools.partial(
        pl.kernel,
        mesh=_sc_mesh(),
        out_type=jax.ShapeDtypeStruct((2 * NP, D), jnp.float32),
        scratch_types=[
            pltpu.VMEM((_CH,), jnp.int32),
            pltpu.VMEM((_CH,), jnp.int32),
            pltpu.VMEM((_CH, D), jnp.float32),
            pltpu.VMEM_SHARED((NP, D), jnp.float32),
            pltpu.SemaphoreType.DMA,
        ],
    )
    def agg_kernel(tab_hbm, src_hbm, dst_hbm, zeros_hbm, out_hbm,
                   si_v, di_v, rows_v, acc_sh, sem):
        cid = lax.axis_index("c")
        sid = lax.axis_index("s")
        r0 = sid * rows_pt
        pltpu.sync_copy(
            zeros_hbm.at[pl.ds(r0, rows_pt)], acc_sh.at[pl.ds(r0, rows_pt)]
        )
        plsc.subcore_barrier()

        def body(i, carry):
            base = cid * E + sid * epc + i * _CH
            pltpu.sync_copy(src_hbm.at[pl.ds(base, _CH)], si_v)
            pltpu.sync_copy(dst_hbm.at[pl.ds(base, _CH)], di_v)
            pltpu.async_copy(tab_hbm.at[si_v], rows_v, sem).wait()
            pltpu.sync_copy(rows_v, acc_sh.at[di_v], add=True)
            return carry

        lax.fori_loop(0, n_chunks, body, 0)
        plsc.subcore_barrier()
        pltpu.sync_copy(
            acc_sh.at[pl.ds(r0, rows_pt)],
            out_hbm.at[pl.ds(cid * NP + r0, rows_pt)],
        )

    return agg_kernel


def _make_state_gather_kernel(TN, B, D):
    bpw = B // (_NC * _NS)

    @functools.partial(
        pl.kernel,
        mesh=_sc_mesh(),
        out_type=jax.ShapeDtypeStruct((B, D), jnp.float32),
        scratch_types=[
            pltpu.VMEM((bpw,), jnp.int32),
            pltpu.VMEM((bpw, D), jnp.float32),
            pltpu.SemaphoreType.DMA,
        ],
    )
    def gather_kernel(tab_hbm, idx_hbm, out_hbm, idx_v, rows_v, sem):
        wid = lax.axis_index("s") * _NC + lax.axis_index("c")
        base = wid * bpw
        pltpu.sync_copy(idx_hbm.at[pl.ds(base, bpw)], idx_v)
        pltpu.async_copy(tab_hbm.at[idx_v], rows_v, sem).wait()
        pltpu.sync_copy(rows_v, out_hbm.at[pl.ds(base, bpw)])

    return gather_kernel


def _prep_body(deg_ref, x_ref, dinv_ref, y0_ref):
    dinv = lax.rsqrt(deg_ref[:, 0:1])
    dinv_ref[:, :] = dinv
    y0_ref[:, :] = x_ref[:, :] * dinv


def _mid_body(agg_ref, y0_ref, dinv_ref, w1_ref, b1_ref, w2_ref, u_ref):
    dinv = dinv_ref[:, :]
    ax = (agg_ref[:, :] + y0_ref[:, :]) * dinv
    h1 = jnp.maximum(
        jnp.dot(ax, w1_ref[:, :], preferred_element_type=jnp.float32) + b1_ref[:],
        0.0,
    )
    u_ref[:, :] = (
        jnp.dot(h1, w2_ref[:, :], preferred_element_type=jnp.float32)
        * dinv
    )


def _final_body(agg_ref, u_ref, dinv_ref, b2_ref, g_ref):
    dinv = dinv_ref[:, :]
    g_ref[:, :] = dinv * (agg_ref[:, :] + u_ref[:, :]) + b2_ref[:]


def _head_body(gx_ref, gy_ref, wh_ref, bh_ref, wp_ref, bp_ref, out_ref):
    z = gx_ref[:, :] * gy_ref[:, :]
    o = jnp.maximum(
        jnp.dot(z, wh_ref[:, :], preferred_element_type=jnp.float32) + bh_ref[:],
        0.0,
    )
    p = jnp.dot(o, wp_ref[:, :], preferred_element_type=jnp.float32) + bp_ref[:]
    m = jnp.max(p, axis=1, keepdims=True)
    e = jnp.exp(p - m)
    out_ref[:, :] = e / jnp.sum(e, axis=1, keepdims=True)


def kernel(x_a, x_b, W1, b1, W2, b2, Wh, bh, Wp, bp,
           edge_index_a, edge_index_b, states):
    N, D = x_a.shape
    E = edge_index_a.shape[1]
    S = states.shape[0]
    H = W1.shape[1]
    NP = ((N + 8 * _NS - 1) // (8 * _NS)) * (8 * _NS)
    TN = 2 * NP

    X = jnp.zeros((TN, D), jnp.float32)
    X = lax.dynamic_update_slice(X, x_a, (0, 0))
    X = lax.dynamic_update_slice(X, x_b, (NP, 0))
    src_all = jnp.concatenate([edge_index_a[0], edge_index_b[0] + NP])
    dst_all = jnp.concatenate([edge_index_a[1], edge_index_b[1]])
    zerosD = jnp.zeros((NP, D), jnp.float32)
    ones16 = jnp.ones((_CH, 16), jnp.float32)
    ones_init = jnp.ones((NP, 16), jnp.float32)
    sidx = jnp.concatenate([states[:, 0], NP + states[:, 1]])

    deg16 = _make_deg_kernel(NP, E)(dst_all, ones16, ones_init)

    R = 1024
    grid = (TN // R,)
    dinv, Y0 = pl.pallas_call(
        _prep_body,
        grid=grid,
        in_specs=[
            pl.BlockSpec((R, 16), lambda i: (i, 0)),
            pl.BlockSpec((R, D), lambda i: (i, 0)),
        ],
        out_specs=[
            pl.BlockSpec((R, 1), lambda i: (i, 0)),
            pl.BlockSpec((R, D), lambda i: (i, 0)),
        ],
        out_shape=[
            jax.ShapeDtypeStruct((TN, 1), jnp.float32),
            jax.ShapeDtypeStruct((TN, D), jnp.float32),
        ],
    )(deg16, X)

    agg_kernel = _make_agg_kernel(NP, E, D)

    agg1 = agg_kernel(Y0, src_all, dst_all, zerosD)

    U = pl.pallas_call(
        _mid_body,
        grid=grid,
        in_specs=[
            pl.BlockSpec((R, D), lambda i: (i, 0)),
            pl.BlockSpec((R, D), lambda i: (i, 0)),
            pl.BlockSpec((R, 1), lambda i: (i, 0)),
            pl.BlockSpec((D, H), lambda i: (0, 0)),
            pl.BlockSpec((H,), lambda i: (0,)),
            pl.BlockSpec((H, D), lambda i: (0, 0)),
        ],
        out_specs=pl.BlockSpec((R, D), lambda i: (i, 0)),
        out_shape=jax.ShapeDtypeStruct((TN, D), jnp.float32),
    )(agg1, Y0, dinv, W1, b1, W2)

    agg2 = agg_kernel(U, src_all, dst_all, zerosD)

    G = pl.pallas_call(
        _final_body,
        grid=grid,
        in_specs=[
            pl.BlockSpec((R, D), lambda i: (i, 0)),
            pl.BlockSpec((R, D), lambda i: (i, 0)),
            pl.BlockSpec((R, 1), lambda i: (i, 0)),
            pl.BlockSpec((D,), lambda i: (0,)),
        ],
        out_specs=pl.BlockSpec((R, D), lambda i: (i, 0)),
        out_shape=jax.ShapeDtypeStruct((TN, D), jnp.float32),
    )(agg2, U, dinv, b2)

    gxy = _make_state_gather_kernel(TN, 2 * S, D)(G, sidx)

    policy = pl.pallas_call(
        _head_body,
        out_shape=jax.ShapeDtypeStruct((S, 2), jnp.float32),
    )(gxy[:S], gxy[S:], Wh, bh, Wp, bp)
    return policy
